# baseline (speedup 1.0000x reference)
import functools

import jax
import jax.numpy as jnp
from jax import lax
from jax.experimental import pallas as pl
from jax.experimental.pallas import tpu as pltpu
from jax.experimental.pallas import tpu_sc as plsc

D_MODEL = 1024
D_FF = 2752
N_EXPERTS = 8
T = 2048
BLK = 512
NB = T * 2 // BLK + N_EXPERTS
R = NB * BLK
CHUNK = 512


def _fiota(shape, dim):
    return lax.broadcasted_iota(jnp.int32, shape, dim).astype(jnp.float32)


def _router_body(x_ref, gw_ref, pos_ref, wts_ref, meta_ref):
    x = x_ref[...]
    gw = gw_ref[...]
    logits = lax.dot_general(gw, x, (((1,), (1,)), ((), ())),
                             preferred_element_type=jnp.float32)
    m = jnp.max(logits, axis=0, keepdims=True)
    p = jnp.exp(logits - m)
    probs = p / jnp.sum(p, axis=0, keepdims=True)

    e_iota = _fiota( (N_EXPERTS, T), 0)
    m1 = jnp.max(probs, axis=0, keepdims=True)
    i1 = jnp.min(jnp.where(probs == m1, e_iota, float(N_EXPERTS)),
                 axis=0, keepdims=True)
    oh1 = (e_iota == i1).astype(jnp.float32)
    probs2 = jnp.where(oh1 > 0.0, -jnp.inf, probs)
    m2 = jnp.max(probs2, axis=0, keepdims=True)
    i2 = jnp.min(jnp.where(probs2 == m2, e_iota, float(N_EXPERTS)),
                 axis=0, keepdims=True)
    oh2 = (e_iota == i2).astype(jnp.float32)

    denom = m1 + m2 + 1e-6
    w1 = m1 / denom
    w2 = m2 / denom

    su = (_fiota( (CHUNK, CHUNK), 0)
          < _fiota( (CHUNK, CHUNK), 1)
          ).astype(jnp.float32)

    def excl_cumsum(oh, carry):
        parts = []
        for c in range(T // CHUNK):
            blk = oh[:, c * CHUNK:(c + 1) * CHUNK]
            parts.append(lax.dot_general(blk, su, (((1,), (0,)), ((), ())),
                                         preferred_element_type=jnp.float32)
                         + carry)
            carry = carry + jnp.sum(blk, axis=1, keepdims=True)
        return jnp.concatenate(parts, axis=1), carry

    zero8 = jnp.zeros((N_EXPERTS, 1), jnp.float32)
    rank0, cnt1 = excl_cumsum(oh1, zero8)
    rank1, counts = excl_cumsum(oh2, cnt1)

    padded = jnp.ceil(counts / BLK) * BLK
    sl8 = (_fiota( (N_EXPERTS, N_EXPERTS), 1)
           < _fiota( (N_EXPERTS, N_EXPERTS), 0)
           ).astype(jnp.float32)
    offs = lax.dot_general(sl8, padded, (((1,), (0,)), ((), ())),
                           preferred_element_type=jnp.float32)
    ends = offs + padded

    pos0 = jnp.sum(oh1 * (offs + rank0), axis=0, keepdims=True)
    pos1 = jnp.sum(oh2 * (offs + rank1), axis=0, keepdims=True)

    lane = _fiota( (1, 128), 1)
    v = jnp.sum((ends <= lane * BLK).astype(jnp.float32), axis=0,
                keepdims=True)
    last_ne = jnp.max(
        jnp.where(counts > 0.0,
                  _fiota( (N_EXPERTS, 1), 0), 0.0))
    be = jnp.minimum(v, last_ne)
    n_act = ends[N_EXPERTS - 1:N_EXPERTS, 0:1] / BLK

    diag = _fiota((8, 128), 0) == _fiota((8, 128), 1)
    bo_row = jnp.sum(jnp.where(diag, offs / BLK, 0.0), axis=0, keepdims=True)
    bo_row = bo_row + (_fiota((1, 128), 1) == 8.0) * n_act

    row2k = _fiota( (8, T), 0)
    pos_ref[...] = ((row2k == 0.0) * pos0 + (row2k == 1.0) * pos1
                    ).astype(jnp.int32)
    wts_ref[...] = (row2k == 0.0) * w1 + (row2k == 1.0) * w2
    row_m = _fiota( (8, 128), 0)
    meta_ref[...] = ((row_m == 0.0) * be + (row_m == 1.0) * n_act
                     + (row_m == 2.0) * bo_row).astype(jnp.int32)


def _router(x2, gate_w):
    return pl.pallas_call(
        _router_body,
        out_shape=(
            jax.ShapeDtypeStruct((8, T), jnp.int32),
            jax.ShapeDtypeStruct((8, T), jnp.float32),
            jax.ShapeDtypeStruct((8, 128), jnp.int32),
        ),
    )(x2, gate_w)


FC = D_FF // 2


def _gateup_body(be_ref, bo_ref, x_ref, wg_hbm, wu_hbm, h_ref,
                 wgf, wuf, wgs, wus, slot_ref, sem_g, sem_u):
    f = pl.program_id(0)
    b = pl.program_id(1)
    na = bo_ref[8]

    def cp(hbm, buf, sem, e, s):
        return pltpu.make_async_copy(
            hbm.at[e, pl.ds(f * FC, FC), :], buf.at[s], sem.at[s])

    @pl.when(b < na)
    def _():
        e = be_ref[b]
        chg = jnp.logical_or(b == 0, e != be_ref[jnp.maximum(b - 1, 0)])
        nxt_b = jnp.minimum(bo_ref[jnp.minimum(e + 1, N_EXPERTS)], NB - 1)
        e_next = be_ref[nxt_b]

        @pl.when(chg)
        def _():
            @pl.when(b == 0)
            def _():
                slot_ref[0] = 0
                cp(wg_hbm, wgf, sem_g, e, 0).start()
                cp(wu_hbm, wuf, sem_u, e, 0).start()

            @pl.when(b > 0)
            def _():
                slot_ref[0] = 1 - slot_ref[0]

            s = slot_ref[0]
            cp(wg_hbm, wgf, sem_g, e, s).wait()
            cp(wu_hbm, wuf, sem_u, e, s).wait()
            wgs[...] = wgf[s].astype(jnp.bfloat16)
            wus[...] = wuf[s].astype(jnp.bfloat16)

            @pl.when(e_next != e)
            def _():
                cp(wg_hbm, wgf, sem_g, e_next, 1 - s).start()
                cp(wu_hbm, wuf, sem_u, e_next, 1 - s).start()

        xb = x_ref[...].astype(jnp.bfloat16)
        g = lax.dot_general(wgs[...], xb, (((1,), (1,)), ((), ())),
                            preferred_element_type=jnp.float32)
        u = lax.dot_general(wus[...], xb, (((1,), (1,)), ((), ())),
                            preferred_element_type=jnp.float32)
        h = g / (1.0 + jnp.exp(-g)) * u
        h_ref[...] = h.astype(jnp.bfloat16)


def _down_body(be_ref, bo_ref, h_ref, wd_hbm, y_ref,
               wdf, wds, slot_ref, sem_d):
    b = pl.program_id(0)
    na = bo_ref[8]

    def cp(e, s):
        return pltpu.make_async_copy(wd_hbm.at[e], wdf.at[s], sem_d.at[s])

    @pl.when(b < na)
    def _():
        e = be_ref[b]
        chg = jnp.logical_or(b == 0, e != be_ref[jnp.maximum(b - 1, 0)])
        nxt_b = jnp.minimum(bo_ref[jnp.minimum(e + 1, N_EXPERTS)], NB - 1)
        e_next = be_ref[nxt_b]

        @pl.when(chg)
        def _():
            @pl.when(b == 0)
            def _():
                slot_ref[0] = 0
                cp(e, 0).start()

            @pl.when(b > 0)
            def _():
                slot_ref[0] = 1 - slot_ref[0]

            s = slot_ref[0]
            cp(e, s).wait()
            wds[...] = wdf[s].astype(jnp.bfloat16)

            @pl.when(e_next != e)
            def _():
                cp(e_next, 1 - s).start()

        y_ref[...] = lax.dot_general(h_ref[...], wds[...],
                                     (((0,), (1,)), ((), ())),
                                     preferred_element_type=jnp.float32)


def _experts(be, bo, xg, wg, wu, wd):
    gs_a = pltpu.PrefetchScalarGridSpec(
        num_scalar_prefetch=2,
        grid=(2, NB),
        in_specs=[
            pl.BlockSpec((BLK, D_MODEL), lambda f, b, be, bo: (b, 0)),
            pl.BlockSpec(memory_space=pl.ANY),
            pl.BlockSpec(memory_space=pl.ANY),
        ],
        out_specs=pl.BlockSpec((FC, BLK), lambda f, b, be, bo: (f, b)),
        scratch_shapes=[
            pltpu.VMEM((2, FC, D_MODEL), jnp.float32),
            pltpu.VMEM((2, FC, D_MODEL), jnp.float32),
            pltpu.VMEM((FC, D_MODEL), jnp.bfloat16),
            pltpu.VMEM((FC, D_MODEL), jnp.bfloat16),
            pltpu.SMEM((1,), jnp.int32),
            pltpu.SemaphoreType.DMA((2,)),
            pltpu.SemaphoreType.DMA((2,)),
        ],
    )
    ht = pl.pallas_call(
        _gateup_body,
        grid_spec=gs_a,
        out_shape=jax.ShapeDtypeStruct((D_FF, R), jnp.bfloat16),
    )(be, bo, xg, wg, wu)

    gs_b = pltpu.PrefetchScalarGridSpec(
        num_scalar_prefetch=2,
        grid=(NB,),
        in_specs=[
            pl.BlockSpec((D_FF, BLK), lambda b, be, bo: (0, b)),
            pl.BlockSpec(memory_space=pl.ANY),
        ],
        out_specs=pl.BlockSpec((BLK, D_MODEL), lambda b, be, bo: (b, 0)),
        scratch_shapes=[
            pltpu.VMEM((2, D_MODEL, D_FF), jnp.float32),
            pltpu.VMEM((D_MODEL, D_FF), jnp.bfloat16),
            pltpu.SMEM((1,), jnp.int32),
            pltpu.SemaphoreType.DMA((2,)),
        ],
    )
    return pl.pallas_call(
        _down_body,
        grid_spec=gs_b,
        out_shape=jax.ShapeDtypeStruct((R, D_MODEL), jnp.float32),
    )(be, bo, ht, wd)


def _combine_body(r0_ref, r1_ref, w0_ref, w1_ref, out_ref):
    out_ref[...] = (w0_ref[...] * r0_ref[...] + w1_ref[...] * r1_ref[...])


def _combine(r0, r1, w0, w1):
    cb = 256
    return pl.pallas_call(
        _combine_body,
        grid=(T // cb,),
        in_specs=[
            pl.BlockSpec((cb, D_MODEL), lambda i: (i, 0)),
            pl.BlockSpec((cb, D_MODEL), lambda i: (i, 0)),
            pl.BlockSpec((cb, 1), lambda i: (i, 0)),
            pl.BlockSpec((cb, 1), lambda i: (i, 0)),
        ],
        out_specs=pl.BlockSpec((cb, D_MODEL), lambda i: (i, 0)),
        out_shape=jax.ShapeDtypeStruct((T, D_MODEL), jnp.float32),
    )(r0, r1, w0, w1)


NW = 32
TPW = T // NW

_SC_MESH = plsc.VectorSubcoreMesh(core_axis_name="c", subcore_axis_name="s",
                                  num_cores=2, num_subcores=16)
_SC_SCRATCH = [
    pltpu.VMEM((TPW,), jnp.int32),
    pltpu.VMEM((TPW, D_MODEL), jnp.float32),
    pltpu.SemaphoreType.DMA,
]


@functools.partial(
    pl.kernel,
    out_type=jax.ShapeDtypeStruct((R, D_MODEL), jnp.float32),
    mesh=_SC_MESH,
    scratch_types=_SC_SCRATCH,
)
def _dispatch(x_hbm, posf_hbm, xg_hbm, idx_v, rows_v, sem):
    wid = lax.axis_index("s") * 2 + lax.axis_index("c")
    base = wid * TPW
    pltpu.sync_copy(x_hbm.at[pl.ds(base, TPW)], rows_v)
    for k in range(2):
        pltpu.sync_copy(posf_hbm.at[pl.ds(k * T + base, TPW)], idx_v)
        pltpu.async_copy(rows_v, xg_hbm.at[idx_v], sem).wait()


@functools.partial(
    pl.kernel,
    out_type=(jax.ShapeDtypeStruct((T, D_MODEL), jnp.float32),
              jax.ShapeDtypeStruct((T, D_MODEL), jnp.float32)),
    mesh=_SC_MESH,
    scratch_types=_SC_SCRATCH,
)
def _gather2(y_hbm, posf_hbm, r0_hbm, r1_hbm, idx_v, rows_v, sem):
    wid = lax.axis_index("s") * 2 + lax.axis_index("c")
    base = wid * TPW
    for k, out in enumerate((r0_hbm, r1_hbm)):
        pltpu.sync_copy(posf_hbm.at[pl.ds(k * T + base, TPW)], idx_v)
        pltpu.async_copy(y_hbm.at[idx_v], rows_v, sem).wait()
        pltpu.sync_copy(rows_v, out.at[pl.ds(base, TPW)])


def kernel(x, gate_w, expert_gate, expert_up, expert_down):
    orig_shape = x.shape
    x2 = x.reshape(T, D_MODEL)
    pos8, wts8, meta = _router(x2, gate_w)
    posf = pos8[:2].reshape(-1)
    be = meta[0, :NB]
    bo = meta[2, :N_EXPERTS + 1]
    xg = _dispatch(x2, posf)
    y = _experts(be, bo, xg, expert_gate, expert_up, expert_down)
    r0, r1 = _gather2(y, posf)
    w0 = wts8[0].reshape(T, 1)
    w1 = wts8[1].reshape(T, 1)
    return _combine(r0, r1, w0, w1).reshape(orig_shape)


# speedup vs baseline: 1.0339x; 1.0339x over previous
import functools

import jax
import jax.numpy as jnp
from jax import lax
from jax.experimental import pallas as pl
from jax.experimental.pallas import tpu as pltpu
from jax.experimental.pallas import tpu_sc as plsc

D_MODEL = 1024
D_FF = 2752
N_EXPERTS = 8
T = 2048
BLK = 256
NB = T * 2 // BLK + N_EXPERTS
R = NB * BLK
CHUNK = 512


def _fiota(shape, dim):
    return lax.broadcasted_iota(jnp.int32, shape, dim).astype(jnp.float32)


def _router_body(x_ref, gw_ref, pos_ref, wts_ref, meta_ref):
    x = x_ref[...]
    gw = gw_ref[...]
    logits = lax.dot_general(gw, x, (((1,), (1,)), ((), ())),
                             preferred_element_type=jnp.float32)
    m = jnp.max(logits, axis=0, keepdims=True)
    p = jnp.exp(logits - m)
    probs = p / jnp.sum(p, axis=0, keepdims=True)

    e_iota = _fiota( (N_EXPERTS, T), 0)
    m1 = jnp.max(probs, axis=0, keepdims=True)
    i1 = jnp.min(jnp.where(probs == m1, e_iota, float(N_EXPERTS)),
                 axis=0, keepdims=True)
    oh1 = (e_iota == i1).astype(jnp.float32)
    probs2 = jnp.where(oh1 > 0.0, -jnp.inf, probs)
    m2 = jnp.max(probs2, axis=0, keepdims=True)
    i2 = jnp.min(jnp.where(probs2 == m2, e_iota, float(N_EXPERTS)),
                 axis=0, keepdims=True)
    oh2 = (e_iota == i2).astype(jnp.float32)

    denom = m1 + m2 + 1e-6
    w1 = m1 / denom
    w2 = m2 / denom

    su = (_fiota( (CHUNK, CHUNK), 0)
          < _fiota( (CHUNK, CHUNK), 1)
          ).astype(jnp.float32)

    def excl_cumsum(oh, carry):
        parts = []
        for c in range(T // CHUNK):
            blk = oh[:, c * CHUNK:(c + 1) * CHUNK]
            parts.append(lax.dot_general(blk, su, (((1,), (0,)), ((), ())),
                                         preferred_element_type=jnp.float32)
                         + carry)
            carry = carry + jnp.sum(blk, axis=1, keepdims=True)
        return jnp.concatenate(parts, axis=1), carry

    zero8 = jnp.zeros((N_EXPERTS, 1), jnp.float32)
    rank0, cnt1 = excl_cumsum(oh1, zero8)
    rank1, counts = excl_cumsum(oh2, cnt1)

    padded = jnp.ceil(counts / BLK) * BLK
    sl8 = (_fiota( (N_EXPERTS, N_EXPERTS), 1)
           < _fiota( (N_EXPERTS, N_EXPERTS), 0)
           ).astype(jnp.float32)
    offs = lax.dot_general(sl8, padded, (((1,), (0,)), ((), ())),
                           preferred_element_type=jnp.float32)
    ends = offs + padded

    pos0 = jnp.sum(oh1 * (offs + rank0), axis=0, keepdims=True)
    pos1 = jnp.sum(oh2 * (offs + rank1), axis=0, keepdims=True)

    lane = _fiota( (1, 128), 1)
    v = jnp.sum((ends <= lane * BLK).astype(jnp.float32), axis=0,
                keepdims=True)
    last_ne = jnp.max(
        jnp.where(counts > 0.0,
                  _fiota( (N_EXPERTS, 1), 0), 0.0))
    be = jnp.minimum(v, last_ne)
    n_act = ends[N_EXPERTS - 1:N_EXPERTS, 0:1] / BLK

    diag = _fiota((8, 128), 0) == _fiota((8, 128), 1)
    bo_row = jnp.sum(jnp.where(diag, offs / BLK, 0.0), axis=0, keepdims=True)
    bo_row = bo_row + (_fiota((1, 128), 1) == 8.0) * n_act

    row2k = _fiota( (8, T), 0)
    pos_ref[...] = ((row2k == 0.0) * pos0 + (row2k == 1.0) * pos1
                    ).astype(jnp.int32)
    wts_ref[...] = (row2k == 0.0) * w1 + (row2k == 1.0) * w2
    row_m = _fiota( (8, 128), 0)
    meta_ref[...] = ((row_m == 0.0) * be + (row_m == 1.0) * n_act
                     + (row_m == 2.0) * bo_row).astype(jnp.int32)


def _router(x2, gate_w):
    return pl.pallas_call(
        _router_body,
        out_shape=(
            jax.ShapeDtypeStruct((8, T), jnp.int32),
            jax.ShapeDtypeStruct((8, T), jnp.float32),
            jax.ShapeDtypeStruct((8, 128), jnp.int32),
        ),
    )(x2, gate_w)


FC = D_FF // 2


def _gateup_body(be_ref, bo_ref, x_ref, wg_hbm, wu_hbm, h_ref,
                 wgf, wuf, wgs, wus, slot_ref, sem_g, sem_u):
    f = pl.program_id(0)
    b = pl.program_id(1)
    na = bo_ref[8]

    def cp(hbm, buf, sem, e, s):
        return pltpu.make_async_copy(
            hbm.at[e, pl.ds(f * FC, FC), :], buf.at[s], sem.at[s])

    @pl.when(b < na)
    def _():
        e = be_ref[b]
        chg = jnp.logical_or(b == 0, e != be_ref[jnp.maximum(b - 1, 0)])
        nxt_b = jnp.minimum(bo_ref[jnp.minimum(e + 1, N_EXPERTS)], NB - 1)
        e_next = be_ref[nxt_b]

        @pl.when(chg)
        def _():
            @pl.when(b == 0)
            def _():
                slot_ref[0] = 0
                cp(wg_hbm, wgf, sem_g, e, 0).start()
                cp(wu_hbm, wuf, sem_u, e, 0).start()

            @pl.when(b > 0)
            def _():
                slot_ref[0] = 1 - slot_ref[0]

            s = slot_ref[0]
            cp(wg_hbm, wgf, sem_g, e, s).wait()
            cp(wu_hbm, wuf, sem_u, e, s).wait()
            wgs[...] = wgf[s].astype(jnp.bfloat16)
            wus[...] = wuf[s].astype(jnp.bfloat16)

            @pl.when(e_next != e)
            def _():
                cp(wg_hbm, wgf, sem_g, e_next, 1 - s).start()
                cp(wu_hbm, wuf, sem_u, e_next, 1 - s).start()

        xb = x_ref[...].astype(jnp.bfloat16)
        g = lax.dot_general(wgs[...], xb, (((1,), (1,)), ((), ())),
                            preferred_element_type=jnp.float32)
        u = lax.dot_general(wus[...], xb, (((1,), (1,)), ((), ())),
                            preferred_element_type=jnp.float32)
        h = g / (1.0 + jnp.exp(-g)) * u
        h_ref[...] = h.astype(jnp.bfloat16)


def _down_body(be_ref, bo_ref, h_ref, wd_hbm, y_ref,
               wdf, wds, slot_ref, sem_d):
    b = pl.program_id(0)
    na = bo_ref[8]

    def cp(e, s):
        return pltpu.make_async_copy(wd_hbm.at[e], wdf.at[s], sem_d.at[s])

    @pl.when(b < na)
    def _():
        e = be_ref[b]
        chg = jnp.logical_or(b == 0, e != be_ref[jnp.maximum(b - 1, 0)])
        nxt_b = jnp.minimum(bo_ref[jnp.minimum(e + 1, N_EXPERTS)], NB - 1)
        e_next = be_ref[nxt_b]

        @pl.when(chg)
        def _():
            @pl.when(b == 0)
            def _():
                slot_ref[0] = 0
                cp(e, 0).start()

            @pl.when(b > 0)
            def _():
                slot_ref[0] = 1 - slot_ref[0]

            s = slot_ref[0]
            cp(e, s).wait()
            wds[...] = wdf[s].astype(jnp.bfloat16)

            @pl.when(e_next != e)
            def _():
                cp(e_next, 1 - s).start()

        y_ref[...] = lax.dot_general(h_ref[...], wds[...],
                                     (((0,), (1,)), ((), ())),
                                     preferred_element_type=jnp.float32)


def _experts(be, bo, xg, wg, wu, wd):
    gs_a = pltpu.PrefetchScalarGridSpec(
        num_scalar_prefetch=2,
        grid=(2, NB),
        in_specs=[
            pl.BlockSpec((BLK, D_MODEL), lambda f, b, be, bo: (b, 0)),
            pl.BlockSpec(memory_space=pl.ANY),
            pl.BlockSpec(memory_space=pl.ANY),
        ],
        out_specs=pl.BlockSpec((FC, BLK), lambda f, b, be, bo: (f, b)),
        scratch_shapes=[
            pltpu.VMEM((2, FC, D_MODEL), jnp.float32),
            pltpu.VMEM((2, FC, D_MODEL), jnp.float32),
            pltpu.VMEM((FC, D_MODEL), jnp.bfloat16),
            pltpu.VMEM((FC, D_MODEL), jnp.bfloat16),
            pltpu.SMEM((1,), jnp.int32),
            pltpu.SemaphoreType.DMA((2,)),
            pltpu.SemaphoreType.DMA((2,)),
        ],
    )
    ht = pl.pallas_call(
        _gateup_body,
        grid_spec=gs_a,
        out_shape=jax.ShapeDtypeStruct((D_FF, R), jnp.bfloat16),
    )(be, bo, xg, wg, wu)

    gs_b = pltpu.PrefetchScalarGridSpec(
        num_scalar_prefetch=2,
        grid=(NB,),
        in_specs=[
            pl.BlockSpec((D_FF, BLK), lambda b, be, bo: (0, b)),
            pl.BlockSpec(memory_space=pl.ANY),
        ],
        out_specs=pl.BlockSpec((BLK, D_MODEL), lambda b, be, bo: (b, 0)),
        scratch_shapes=[
            pltpu.VMEM((2, D_MODEL, D_FF), jnp.float32),
            pltpu.VMEM((D_MODEL, D_FF), jnp.bfloat16),
            pltpu.SMEM((1,), jnp.int32),
            pltpu.SemaphoreType.DMA((2,)),
        ],
    )
    return pl.pallas_call(
        _down_body,
        grid_spec=gs_b,
        out_shape=jax.ShapeDtypeStruct((R, D_MODEL), jnp.float32),
    )(be, bo, ht, wd)


def _combine_body(r0_ref, r1_ref, w0_ref, w1_ref, out_ref):
    out_ref[...] = (w0_ref[...] * r0_ref[...] + w1_ref[...] * r1_ref[...])


def _combine(r0, r1, w0, w1):
    cb = 256
    return pl.pallas_call(
        _combine_body,
        grid=(T // cb,),
        in_specs=[
            pl.BlockSpec((cb, D_MODEL), lambda i: (i, 0)),
            pl.BlockSpec((cb, D_MODEL), lambda i: (i, 0)),
            pl.BlockSpec((cb, 1), lambda i: (i, 0)),
            pl.BlockSpec((cb, 1), lambda i: (i, 0)),
        ],
        out_specs=pl.BlockSpec((cb, D_MODEL), lambda i: (i, 0)),
        out_shape=jax.ShapeDtypeStruct((T, D_MODEL), jnp.float32),
    )(r0, r1, w0, w1)


NW = 32
TPW = T // NW

_SC_MESH = plsc.VectorSubcoreMesh(core_axis_name="c", subcore_axis_name="s",
                                  num_cores=2, num_subcores=16)
_SC_SCRATCH = [
    pltpu.VMEM((TPW,), jnp.int32),
    pltpu.VMEM((TPW, D_MODEL), jnp.float32),
    pltpu.SemaphoreType.DMA,
]


@functools.partial(
    pl.kernel,
    out_type=jax.ShapeDtypeStruct((R, D_MODEL), jnp.float32),
    mesh=_SC_MESH,
    scratch_types=_SC_SCRATCH,
)
def _dispatch(x_hbm, posf_hbm, xg_hbm, idx_v, rows_v, sem):
    wid = lax.axis_index("s") * 2 + lax.axis_index("c")
    base = wid * TPW
    pltpu.sync_copy(x_hbm.at[pl.ds(base, TPW)], rows_v)
    for k in range(2):
        pltpu.sync_copy(posf_hbm.at[pl.ds(k * T + base, TPW)], idx_v)
        pltpu.async_copy(rows_v, xg_hbm.at[idx_v], sem).wait()


@functools.partial(
    pl.kernel,
    out_type=(jax.ShapeDtypeStruct((T, D_MODEL), jnp.float32),
              jax.ShapeDtypeStruct((T, D_MODEL), jnp.float32)),
    mesh=_SC_MESH,
    scratch_types=_SC_SCRATCH,
)
def _gather2(y_hbm, posf_hbm, r0_hbm, r1_hbm, idx_v, rows_v, sem):
    wid = lax.axis_index("s") * 2 + lax.axis_index("c")
    base = wid * TPW
    for k, out in enumerate((r0_hbm, r1_hbm)):
        pltpu.sync_copy(posf_hbm.at[pl.ds(k * T + base, TPW)], idx_v)
        pltpu.async_copy(y_hbm.at[idx_v], rows_v, sem).wait()
        pltpu.sync_copy(rows_v, out.at[pl.ds(base, TPW)])


def kernel(x, gate_w, expert_gate, expert_up, expert_down):
    orig_shape = x.shape
    x2 = x.reshape(T, D_MODEL)
    pos8, wts8, meta = _router(x2, gate_w)
    posf = pos8[:2].reshape(-1)
    be = meta[0, :NB]
    bo = meta[2, :N_EXPERTS + 1]
    xg = _dispatch(x2, posf)
    y = _experts(be, bo, xg, expert_gate, expert_up, expert_down)
    r0, r1 = _gather2(y, posf)
    w0 = wts8[0].reshape(T, 1)
    w1 = wts8[1].reshape(T, 1)
    return _combine(r0, r1, w0, w1).reshape(orig_shape)
